# baseline (speedup 1.0000x reference)
import functools

import jax
import jax.numpy as jnp
from jax import lax
from jax.experimental import pallas as pl
from jax.experimental.pallas import tpu as pltpu
from jax.experimental.pallas import tpu_sc as plsc

BATCH = 16384
EMBED = 64
NUM_CORES = 2
NUM_SUBCORES = 16
NW = NUM_CORES * NUM_SUBCORES
B_PER_W = BATCH // NW
HALF = B_PER_W // 2
CHUNK = 128
NCH = B_PER_W // CHUNK


def _mesh():
    return plsc.VectorSubcoreMesh(core_axis_name="c", subcore_axis_name="s")


def _user_body(idx_hbm, tab, out_u, idx_v, rows_v, sem):
    wid = lax.axis_index("s") * NUM_CORES + lax.axis_index("c")
    base = wid * B_PER_W
    pltpu.sync_copy(idx_hbm.at[wid], idx_v)
    copies = [
        pltpu.async_copy(
            tab.at[idx_v.at[c]], rows_v.at[pl.ds(c * CHUNK, CHUNK)], sem
        )
        for c in range(NCH)
    ]
    for cp in copies:
        cp.wait()
    pltpu.sync_copy(rows_v, out_u.at[pl.ds(base, B_PER_W)])


def _item_body(idx_hbm, item_tab, out_p, out_n, idx_v, rows_v, sem0, sem1):
    wid = lax.axis_index("s") * NUM_CORES + lax.axis_index("c")
    base = wid * B_PER_W
    pltpu.sync_copy(idx_hbm.at[wid], idx_v)
    sems = (sem0, sem1)
    outs = (out_p, out_n)
    for h in range(2):
        for k in range(2):
            def issue(g, carry, sem=sems[k], k=k, h=h):
                vec = idx_v[pl.ds(k * B_PER_W + h * HALF + g * 16, 16)]
                for l in range(16):
                    row = vec[l]
                    pltpu.async_copy(
                        item_tab.at[row], rows_v.at[k, g * 16 + l], sem
                    )
                return carry

            lax.fori_loop(0, HALF // 16, issue, 0)
        for k in range(2):
            pltpu.make_async_copy(
                item_tab.at[pl.ds(0, HALF)], rows_v.at[k], sems[k]
            ).wait()
            pltpu.sync_copy(
                rows_v.at[k], outs[k].at[pl.ds(base + h * HALF, HALF)]
            )


@jax.jit
def _sbpr(idx_u, idx_pn, embed_user, embed_item):
    out = jax.ShapeDtypeStruct((BATCH, EMBED), jnp.float32)
    out_p, out_n = pl.kernel(
        _item_body,
        out_type=(out, out),
        mesh=_mesh(),
        scratch_types=[
            pltpu.VMEM((2 * B_PER_W,), jnp.int32),
            pltpu.VMEM((2, HALF, EMBED), jnp.float32),
            pltpu.SemaphoreType.DMA,
            pltpu.SemaphoreType.DMA,
        ],
    )(idx_pn, embed_item)
    out_u = pl.kernel(
        _user_body,
        out_type=(out,),
        mesh=_mesh(),
        scratch_types=[
            pltpu.VMEM((NCH, CHUNK), jnp.int32),
            pltpu.VMEM((B_PER_W, EMBED), jnp.float32),
            pltpu.SemaphoreType.DMA,
        ],
        compiler_params=pltpu.CompilerParams(use_tc_tiling_on_sc=False),
    )(idx_u, embed_user)[0]
    return out_u, out_p, out_n


def kernel(batch_user, batch_pos_item, batch_neg_item, embed_user, embed_item):
    idx_u = batch_user.reshape(NW, NCH, CHUNK)
    idx_pn = (
        jnp.stack([batch_pos_item, batch_neg_item])
        .reshape(2, NW, B_PER_W)
        .transpose(1, 0, 2)
        .reshape(NW, 2 * B_PER_W)
    )
    return _sbpr(idx_u, idx_pn, embed_user, embed_item)
